# baseline (speedup 1.0000x reference)
import functools

import jax
import jax.numpy as jnp
from jax import lax
from jax.experimental import pallas as pl
from jax.experimental.pallas import tpu as pltpu
from jax.experimental.pallas import tpu_sc as plsc

_N = 16000
_E = 88000
_HEADS = 4
_HID = 128
_TYPES = ('a', 'r', 'n')
_EL = (('a', 'r'), ('n', 'r'), ('n', 'a'), ('a', 'a'), ('r', 'n'),
       ('r', 'a'), ('a', 'a'), ('r', 'r'), ('n', 'n'))
_GROUPS = (('r', (0, 1, 7)), ('a', (2, 3, 5, 6)), ('n', (4, 8)))
_TIDX = {'a': 0, 'r': 1, 'n': 2}
_SRC_RELS = {t: tuple(i for i, e in enumerate(_EL) if e[0] == t)
             for t in _TYPES}
_DST_RELS = {t: tuple(i for i, e in enumerate(_EL) if e[1] == t)
             for t in _TYPES}
_REL_LANES = tuple(
    (_TIDX[st], 2 * _SRC_RELS[st].index(i),
     _TIDX[dt], 6 + 2 * _DST_RELS[dt].index(i))
    for i, (st, dt) in enumerate(_EL))

_NTILE = 16
_TILE_E = 5632
_EPAD = _NTILE * _TILE_E
_K = 176
_NCHUNK = _TILE_E // _K
_SLAB = _N // _NTILE


def _mm_body(x_ref, w_ref, o_ref):
    o_ref[...] = jnp.dot(x_ref[...], w_ref[...],
                         preferred_element_type=jnp.float32)


def _mm(x, w, bm=2000):
    m, k = x.shape
    _, n = w.shape
    return pl.pallas_call(
        _mm_body,
        grid=(m // bm,),
        in_specs=[pl.BlockSpec((bm, k), lambda i: (i, 0)),
                  pl.BlockSpec((k, n), lambda i: (0, 0))],
        out_specs=pl.BlockSpec((bm, n), lambda i: (i, 0)),
        out_shape=jax.ShapeDtypeStruct((m, n), jnp.float32),
    )(x, w)


def _proj_body(fcols, lcols, rcols, h_ref, w_ref, *out_refs):
    big = jnp.dot(h_ref[...], w_ref[...], preferred_element_type=jnp.float32)
    bm = big.shape[0]
    for o_ref, a in zip(out_refs[:-1], fcols):
        f = big[:, a:a + _HID]
        o_ref[...] = jnp.stack([f[:, :64], f[:, 64:]], axis=0)
    zpad = jnp.zeros((bm, 16 - 2 * len(lcols) - 2 * len(rcols)), jnp.float32)
    rows = []
    for cc in (0, 1):
        parts = [big[:, a + 2 * cc:a + 2 * cc + 2] for a in lcols]
        parts += [big[:, a + 2 * cc:a + 2 * cc + 2] for a in rcols]
        parts.append(zpad)
        rows.append(jnp.concatenate(parts, axis=1))
    out_refs[-1][...] = jnp.stack(rows, axis=0)


def _proj(h, w, fcols, lcols, rcols, bm=2000):
    m, k = h.shape
    _, n = w.shape
    out_shapes = [jax.ShapeDtypeStruct((2, m, 64), jnp.float32)
                  for _ in fcols]
    out_specs = [pl.BlockSpec((2, bm, 64), lambda i: (0, i, 0))
                 for _ in fcols]
    out_shapes.append(jax.ShapeDtypeStruct((2, m, 16), jnp.float32))
    out_specs.append(pl.BlockSpec((2, bm, 16), lambda i: (0, i, 0)))
    return pl.pallas_call(
        functools.partial(_proj_body, fcols, lcols, rcols),
        grid=(m // bm,),
        in_specs=[pl.BlockSpec((bm, k), lambda i: (i, 0)),
                  pl.BlockSpec((k, n), lambda i: (0, 0))],
        out_specs=out_specs,
        out_shape=out_shapes,
    )(h, w)


def _ln_body(a_ref, g_ref, b_ref, bias_ref, o_ref):
    a = a_ref[...]
    v = jnp.concatenate([a[0], a[1]], axis=1) + bias_ref[...]
    mu = jnp.mean(v, axis=-1, keepdims=True)
    var = jnp.mean((v - mu) ** 2, axis=-1, keepdims=True)
    v = (v - mu) * lax.rsqrt(var + 1e-5) * g_ref[...] + b_ref[...]
    o_ref[...] = jnp.maximum(v, 0.0)


def _ln_relu(acc, gamma, beta, bias_sum, bm=2000):
    n = _HID
    return pl.pallas_call(
        _ln_body,
        grid=(_N // bm,),
        in_specs=[pl.BlockSpec((2, bm, 64), lambda i: (0, i, 0)),
                  pl.BlockSpec((1, n), lambda i: (0, 0)),
                  pl.BlockSpec((1, n), lambda i: (0, 0)),
                  pl.BlockSpec((1, n), lambda i: (0, 0))],
        out_specs=pl.BlockSpec((bm, n), lambda i: (i, 0)),
        out_shape=jax.ShapeDtypeStruct((_N, n), jnp.float32),
    )(acc, gamma.reshape(1, n), beta.reshape(1, n), bias_sum.reshape(1, n))


def _head_mat(a):
    return (a[:, :, None] * jnp.eye(_HEADS)[:, None, :]).reshape(_HID, _HEADS)


def _lane_bcast(vec, idx):
    return lax.gather(
        vec, idx[:, None],
        lax.GatherDimensionNumbers(offset_dims=(), collapsed_slice_dims=(0,),
                                   start_index_map=(0,)),
        slice_sizes=(1,), mode=lax.GatherScatterMode.PROMISE_IN_BOUNDS)


def _sc_layer_body(*refs):
    es_t = refs[0:9]
    ed_t = refs[9:18]
    pk_t = refs[18:21]
    fs_t = refs[21:30]
    out_hbms = refs[30:33]
    ex_hbm = refs[33]
    (s_sh, out_sh, el0, el1, er0, er1, sx0, sx1, f0, f1,
     srco0, srco1, dsto0, dsto1, dst0, dst1, z16, z64,
     sA0, sA1, sB0, sB1, sF0, sF1, sE0, sE1,
     sIa0, sIb0, sIa1, sIb1) = refs[34:]

    c = lax.axis_index("c")
    s = lax.axis_index("s")
    base_e = s * _TILE_E
    noff = c * _N
    zv = jnp.zeros((16,), jnp.float32)
    idx0 = jnp.zeros((16,), jnp.int32)
    idx1 = jnp.ones((16,), jnp.int32)
    pairs = _NCHUNK // 2

    def zfill(k, _):
        z16[k] = zv
        for q in range(4):
            z64[k, pl.ds(q * 16, 16)] = zv
        return 0
    lax.fori_loop(0, 40, zfill, 0)

    def zero_out(q, _):
        pltpu.sync_copy(z64, out_sh.at[pl.ds(s * _SLAB + q * 40, 40)])
        return 0

    def zero_s(q, _):
        pltpu.sync_copy(z16, s_sh.at[pl.ds(s * _SLAB + q * 40, 40)])
        return 0

    lanes = lax.iota(jnp.int32, 16)

    def make_exrow(elb, erb, sxb, off, idxp, idxq):
        def exrow(i, _):
            for u in range(4):
                k = i * 4 + u
                row = (_lane_bcast(elb[k], idxp) +
                       _lane_bcast(erb[k], idxq))
                row = jnp.where(row > 0, row, 0.2 * row)
                row = jnp.exp(row)
                m = jnp.where(off + k < _E, 1.0, 0.0)
                sxb[k] = row * m
            return 0
        return exrow

    def make_msgrow(exb, srowb, fb):
        def msgrow(i, _):
            for u in range(4):
                k = i * 4 + u
                coef = exb[k] / (srowb[k] + 1e-9)
                c0 = _lane_bcast(coef, idx0)
                c1 = _lane_bcast(coef, idx1)
                for q in range(4):
                    sl = pl.ds(q * 16, 16)
                    fb[k, sl] = fb[k, sl] * (c0 if q < 2 else c1)
            return 0
        return msgrow

    for g, (dt, rels) in enumerate(_GROUPS):
        lax.fori_loop(0, _SLAB // 40, zero_out, 0)
        plsc.subcore_barrier()
        for r in rels:
            sti, lp, dti, lq = _REL_LANES[r]
            es_r, ed_r, el_r, er_r, fs_r = (es_t[r], ed_t[r], pk_t[sti],
                                            pk_t[dti], fs_t[r])
            idxp = jnp.where(lanes < 2, lanes + lp, lp)
            idxq = jnp.where(lanes < 2, lanes + lq, lq)
            lax.fori_loop(0, _SLAB // 40, zero_s, 0)
            plsc.subcore_barrier()

            def idx_issue(off, srco, dstb, sia, sib):
                pltpu.async_copy(es_r.at[pl.ds(off, _K)], srco, sia)
                pltpu.async_copy(ed_r.at[pl.ds(off, _K)], dstb, sib)

            def idx_wait(srco, dstb, sia, sib):
                pltpu.make_async_copy(es_r.at[pl.ds(base_e, _K)], srco,
                                      sia).wait()
                pltpu.make_async_copy(ed_r.at[pl.ds(base_e, _K)], dstb,
                                      sib).wait()

            def g1_issue(srco, dsto, dstb, elb, erb, sa, sb):
                def addoff(g_, _):
                    sl = pl.ds(g_ * 16, 16)
                    srco[sl] = srco[sl] + noff
                    dsto[sl] = dstb[sl] + noff
                    return 0
                lax.fori_loop(0, _K // 16, addoff, 0)
                pltpu.async_copy(el_r.at[srco], elb, sa)
                pltpu.async_copy(er_r.at[dsto], erb, sb)

            def g1_wait(srco, dsto, elb, erb, sa, sb):
                pltpu.make_async_copy(el_r.at[srco], elb, sa).wait()
                pltpu.make_async_copy(er_r.at[dsto], erb, sb).wait()

            idx_issue(base_e, srco0, dst0, sIa0, sIb0)
            idx_issue(base_e + _K, srco1, dst1, sIa1, sIb1)
            idx_wait(srco0, dst0, sIa0, sIb0)
            g1_issue(srco0, dsto0, dst0, el0, er0, sA0, sB0)
            idx_wait(srco1, dst1, sIa1, sIb1)
            g1_issue(srco1, dsto1, dst1, el1, er1, sA1, sB1)

            def p1_pair(i, _):
                off0 = base_e + (2 * i) * _K
                off1 = off0 + _K
                more = i + 1 < pairs
                g1_wait(srco0, dsto0, el0, er0, sA0, sB0)
                lax.fori_loop(0, _K // 4,
                              make_exrow(el0, er0, sx0, off0,
                                         idxp, idxq), 0)
                cpe0 = pltpu.async_copy(sx0, ex_hbm.at[c, pl.ds(off0, _K)],
                                        sE0)
                pltpu.sync_copy(sx0, s_sh.at[dst0], add=True)

                @pl.when(more)
                def _():
                    idx_issue(off0 + 2 * _K, srco0, dst0, sIa0, sIb0)
                g1_wait(srco1, dsto1, el1, er1, sA1, sB1)

                lax.fori_loop(0, _K // 4,
                              make_exrow(el1, er1, sx1, off1,
                                         idxp, idxq), 0)
                cpe1 = pltpu.async_copy(sx1, ex_hbm.at[c, pl.ds(off1, _K)],
                                        sE1)
                pltpu.sync_copy(sx1, s_sh.at[dst1], add=True)

                @pl.when(more)
                def _():
                    idx_issue(off1 + 2 * _K, srco1, dst1, sIa1, sIb1)
                    idx_wait(srco0, dst0, sIa0, sIb0)
                    g1_issue(srco0, dsto0, dst0, el0, er0, sA0, sB0)
                    idx_wait(srco1, dst1, sIa1, sIb1)
                    g1_issue(srco1, dsto1, dst1, el1, er1, sA1, sB1)
                cpe0.wait()
                cpe1.wait()
                return 0
            lax.fori_loop(0, pairs, p1_pair, 0)
            plsc.subcore_barrier()

            def g2_issue(off, srco, dstb, exb, srowb, fb, sa, sb, sf):
                def addoff(g_, _):
                    sl = pl.ds(g_ * 16, 16)
                    srco[sl] = srco[sl] + noff
                    return 0
                lax.fori_loop(0, _K // 16, addoff, 0)
                pltpu.async_copy(ex_hbm.at[c, pl.ds(off, _K)], exb, sa)
                pltpu.async_copy(s_sh.at[dstb], srowb, sb)
                pltpu.async_copy(fs_r.at[srco], fb, sf)

            def g2_wait(srco, dstb, exb, srowb, fb, sa, sb, sf):
                pltpu.make_async_copy(ex_hbm.at[c, pl.ds(base_e, _K)], exb,
                                      sa).wait()
                pltpu.make_async_copy(s_sh.at[dstb], srowb, sb).wait()
                pltpu.make_async_copy(fs_r.at[srco], fb, sf).wait()

            idx_issue(base_e, srco0, dst0, sIa0, sIb0)
            idx_issue(base_e + _K, srco1, dst1, sIa1, sIb1)
            idx_wait(srco0, dst0, sIa0, sIb0)
            g2_issue(base_e, srco0, dst0, el0, er0, f0, sA0, sB0, sF0)
            idx_wait(srco1, dst1, sIa1, sIb1)
            g2_issue(base_e + _K, srco1, dst1, el1, er1, f1, sA1, sB1, sF1)

            def p2_pair(i, _):
                off0 = base_e + (2 * i) * _K
                off1 = off0 + _K
                more = i + 1 < pairs
                g2_wait(srco0, dst0, el0, er0, f0, sA0, sB0, sF0)
                lax.fori_loop(0, _K // 4, make_msgrow(el0, er0, f0), 0)
                pltpu.sync_copy(f0, out_sh.at[dst0], add=True)

                @pl.when(more)
                def _():
                    idx_issue(off0 + 2 * _K, srco0, dst0, sIa0, sIb0)
                g2_wait(srco1, dst1, el1, er1, f1, sA1, sB1, sF1)

                lax.fori_loop(0, _K // 4, make_msgrow(el1, er1, f1), 0)
                pltpu.sync_copy(f1, out_sh.at[dst1], add=True)

                @pl.when(more)
                def _():
                    idx_issue(off1 + 2 * _K, srco1, dst1, sIa1, sIb1)
                    idx_wait(srco0, dst0, sIa0, sIb0)
                    g2_issue(off0 + 2 * _K, srco0, dst0, el0, er0, f0,
                             sA0, sB0, sF0)
                    idx_wait(srco1, dst1, sIa1, sIb1)
                    g2_issue(off1 + 2 * _K, srco1, dst1, el1, er1, f1,
                             sA1, sB1, sF1)
                return 0
            lax.fori_loop(0, pairs, p2_pair, 0)
            plsc.subcore_barrier()

        pltpu.sync_copy(out_sh.at[pl.ds(s * _SLAB, _SLAB)],
                        out_hbms[g].at[c, pl.ds(s * _SLAB, _SLAB)])
        plsc.subcore_barrier()


@functools.lru_cache(maxsize=None)
def _sc_layer_call():
    mesh = plsc.VectorSubcoreMesh(core_axis_name="c", subcore_axis_name="s")
    return pl.kernel(
        _sc_layer_body,
        out_type=[jax.ShapeDtypeStruct((2, _N, 64), jnp.float32)] * 3 +
                 [jax.ShapeDtypeStruct((2, _EPAD, 16), jnp.float32)],
        mesh=mesh,
        compiler_params=pltpu.CompilerParams(use_tc_tiling_on_sc=False),
        scratch_types=[
            pltpu.VMEM_SHARED((_N, 16), jnp.float32),
            pltpu.VMEM_SHARED((_N, 64), jnp.float32),
            pltpu.VMEM((_K, 16), jnp.float32),
            pltpu.VMEM((_K, 16), jnp.float32),
            pltpu.VMEM((_K, 16), jnp.float32),
            pltpu.VMEM((_K, 16), jnp.float32),
            pltpu.VMEM((_K, 16), jnp.float32),
            pltpu.VMEM((_K, 16), jnp.float32),
            pltpu.VMEM((_K, 64), jnp.float32),
            pltpu.VMEM((_K, 64), jnp.float32),
            pltpu.VMEM((_K,), jnp.int32),
            pltpu.VMEM((_K,), jnp.int32),
            pltpu.VMEM((_K,), jnp.int32),
            pltpu.VMEM((_K,), jnp.int32),
            pltpu.VMEM((_K,), jnp.int32),
            pltpu.VMEM((_K,), jnp.int32),
            pltpu.VMEM((40, 16), jnp.float32),
            pltpu.VMEM((40, 64), jnp.float32),
        ] + [pltpu.SemaphoreType.DMA] * 12,
    )


def _edge_phase(fsrc, elerpk, esrc, edst):
    outs = _sc_layer_call()(*esrc, *edst, *elerpk, *fsrc)
    return {dt: outs[g] for g, (dt, _) in enumerate(_GROUPS)}


def kernel(x_assmpt, x_rule, x_non_assmpt,
           e0, e1, e2, e3, e4, e5, e6, e7, e8,
           W_emb_a, b_emb_a, W_emb_r, b_emb_r, W_emb_n, b_emb_n,
           fc_W0, fc_b0, al0, ar0,
           fc_W1, fc_b1, al1, ar1,
           fc_W2, fc_b2, al2, ar2,
           ln_g_a, ln_b_a, cls_W_a, cls_b_a,
           ln_g_r, ln_b_r, cls_W_r, cls_b_r,
           ln_g_n, ln_b_n, cls_W_n, cls_b_n):
    xs = {'a': x_assmpt, 'r': x_rule, 'n': x_non_assmpt}
    wemb = {'a': (W_emb_a, b_emb_a), 'r': (W_emb_r, b_emb_r),
            'n': (W_emb_n, b_emb_n)}
    lng = {'a': (ln_g_a, ln_b_a), 'r': (ln_g_r, ln_b_r), 'n': (ln_g_n, ln_b_n)}
    cls = {'a': (cls_W_a, cls_b_a), 'r': (cls_W_r, cls_b_r),
           'n': (cls_W_n, cls_b_n)}
    fcw = (fc_W0, fc_W1, fc_W2)
    fcb = (fc_b0, fc_b1, fc_b2)
    als = (al0, al1, al2)
    ars = (ar0, ar1, ar2)
    edges = (e0, e1, e2, e3, e4, e5, e6, e7, e8)

    zpad = jnp.zeros((_EPAD - _E,), jnp.int32)
    esrc = [jnp.concatenate([e[0], zpad]) for e in edges]
    edst = [jnp.concatenate([e[1], zpad]) for e in edges]

    h = {t: _mm(xs[t], wemb[t][0].T) + wemb[t][1] for t in _TYPES}

    for l in range(3):
        fsrc, elerpk = [None] * 9, [None] * 3
        for t in _TYPES:
            pieces, fcols, lcols, rcols = [], [], [], []
            w = 0
            for i in _SRC_RELS[t]:
                wt = fcw[l][i].T
                fcols.append(w)
                pieces.append(wt)
                w += _HID
                lcols.append(w)
                pieces.append(wt @ _head_mat(als[l][i]))
                w += _HEADS
            for i in _DST_RELS[t]:
                rcols.append(w)
                pieces.append(fcw[l][i].T @ _head_mat(ars[l][i]))
                w += _HEADS
            outs = _proj(h[t], jnp.concatenate(pieces, axis=1),
                         tuple(fcols), tuple(lcols), tuple(rcols))
            for i, o in zip(_SRC_RELS[t], outs[:-1]):
                fsrc[i] = o.reshape(2 * _N, 64)
            elerpk[_TIDX[t]] = outs[-1].reshape(2 * _N, 16)

        acc = _edge_phase(fsrc, elerpk, esrc, edst)

        bias_sums = {t: jnp.zeros((_HID,), jnp.float32) for t in _TYPES}
        for i, (st, dt) in enumerate(_EL):
            bias_sums[dt] = bias_sums[dt] + fcb[l][i]
        h = {t: _ln_relu(acc[t], lng[t][0], lng[t][1], bias_sums[t])
             for t in _TYPES}

    outs = []
    for t in _TYPES:
        w = jnp.concatenate(
            [cls[t][0].T, jnp.zeros((_HID, 8 - cls[t][0].shape[0]),
                                    jnp.float32)], axis=1)
        o = _mm(h[t], w)[:, :cls[t][0].shape[0]] + cls[t][1]
        outs.append(o)
    return tuple(outs)


# speedup vs baseline: 1.0045x; 1.0045x over previous
import functools

import jax
import jax.numpy as jnp
from jax import lax
from jax.experimental import pallas as pl
from jax.experimental.pallas import tpu as pltpu
from jax.experimental.pallas import tpu_sc as plsc

_N = 16000
_E = 88000
_HEADS = 4
_HID = 128
_TYPES = ('a', 'r', 'n')
_EL = (('a', 'r'), ('n', 'r'), ('n', 'a'), ('a', 'a'), ('r', 'n'),
       ('r', 'a'), ('a', 'a'), ('r', 'r'), ('n', 'n'))
_GROUPS = (('r', (0, 1, 7)), ('a', (2, 3, 5, 6)), ('n', (4, 8)))
_TIDX = {'a': 0, 'r': 1, 'n': 2}
_SRC_RELS = {t: tuple(i for i, e in enumerate(_EL) if e[0] == t)
             for t in _TYPES}
_DST_RELS = {t: tuple(i for i, e in enumerate(_EL) if e[1] == t)
             for t in _TYPES}
_REL_LANES = tuple(
    (_TIDX[st], 2 * _SRC_RELS[st].index(i),
     _TIDX[dt], 6 + 2 * _DST_RELS[dt].index(i))
    for i, (st, dt) in enumerate(_EL))

_NTILE = 16
_TILE_E = 5632
_EPAD = _NTILE * _TILE_E
_K = 176
_NCHUNK = _TILE_E // _K
_SLAB = _N // _NTILE


def _mm_body(x_ref, w_ref, o_ref):
    o_ref[...] = jnp.dot(x_ref[...], w_ref[...],
                         preferred_element_type=jnp.float32)


def _mm(x, w, bm=2000):
    m, k = x.shape
    _, n = w.shape
    return pl.pallas_call(
        _mm_body,
        grid=(m // bm,),
        in_specs=[pl.BlockSpec((bm, k), lambda i: (i, 0)),
                  pl.BlockSpec((k, n), lambda i: (0, 0))],
        out_specs=pl.BlockSpec((bm, n), lambda i: (i, 0)),
        out_shape=jax.ShapeDtypeStruct((m, n), jnp.float32),
    )(x, w)


def _proj_body(fcols, lcols, rcols, h_ref, w_ref, *out_refs):
    big = jnp.dot(h_ref[...], w_ref[...], preferred_element_type=jnp.float32)
    bm = big.shape[0]
    for o_ref, a in zip(out_refs[:-1], fcols):
        f = big[:, a:a + _HID]
        o_ref[...] = jnp.stack([f[:, :64], f[:, 64:]], axis=0)
    zpad = jnp.zeros((bm, 16 - 2 * len(lcols) - 2 * len(rcols)), jnp.float32)
    rows = []
    for cc in (0, 1):
        parts = [big[:, a + 2 * cc:a + 2 * cc + 2] for a in lcols]
        parts += [big[:, a + 2 * cc:a + 2 * cc + 2] for a in rcols]
        parts.append(zpad)
        rows.append(jnp.concatenate(parts, axis=1))
    out_refs[-1][...] = jnp.stack(rows, axis=0)


def _proj(h, w, fcols, lcols, rcols, bm=2000):
    m, k = h.shape
    _, n = w.shape
    out_shapes = [jax.ShapeDtypeStruct((2, m, 64), jnp.float32)
                  for _ in fcols]
    out_specs = [pl.BlockSpec((2, bm, 64), lambda i: (0, i, 0))
                 for _ in fcols]
    out_shapes.append(jax.ShapeDtypeStruct((2, m, 16), jnp.float32))
    out_specs.append(pl.BlockSpec((2, bm, 16), lambda i: (0, i, 0)))
    return pl.pallas_call(
        functools.partial(_proj_body, fcols, lcols, rcols),
        grid=(m // bm,),
        in_specs=[pl.BlockSpec((bm, k), lambda i: (i, 0)),
                  pl.BlockSpec((k, n), lambda i: (0, 0))],
        out_specs=out_specs,
        out_shape=out_shapes,
    )(h, w)


def _ln_body(a_ref, g_ref, b_ref, bias_ref, o_ref):
    a = a_ref[...]
    v = jnp.concatenate([a[0], a[1]], axis=1) + bias_ref[...]
    mu = jnp.mean(v, axis=-1, keepdims=True)
    var = jnp.mean((v - mu) ** 2, axis=-1, keepdims=True)
    v = (v - mu) * lax.rsqrt(var + 1e-5) * g_ref[...] + b_ref[...]
    o_ref[...] = jnp.maximum(v, 0.0)


def _ln_relu(acc, gamma, beta, bias_sum, bm=2000):
    n = _HID
    return pl.pallas_call(
        _ln_body,
        grid=(_N // bm,),
        in_specs=[pl.BlockSpec((2, bm, 64), lambda i: (0, i, 0)),
                  pl.BlockSpec((1, n), lambda i: (0, 0)),
                  pl.BlockSpec((1, n), lambda i: (0, 0)),
                  pl.BlockSpec((1, n), lambda i: (0, 0))],
        out_specs=pl.BlockSpec((bm, n), lambda i: (i, 0)),
        out_shape=jax.ShapeDtypeStruct((_N, n), jnp.float32),
    )(acc, gamma.reshape(1, n), beta.reshape(1, n), bias_sum.reshape(1, n))


def _head_mat(a):
    return (a[:, :, None] * jnp.eye(_HEADS)[:, None, :]).reshape(_HID, _HEADS)


def _lane_bcast(vec, idx):
    return lax.gather(
        vec, idx[:, None],
        lax.GatherDimensionNumbers(offset_dims=(), collapsed_slice_dims=(0,),
                                   start_index_map=(0,)),
        slice_sizes=(1,), mode=lax.GatherScatterMode.PROMISE_IN_BOUNDS)


def _sc_layer_body(*refs):
    es_t = refs[0:9]
    ed_t = refs[9:18]
    pk_t = refs[18:21]
    fs_t = refs[21:30]
    out_hbms = refs[30:33]
    ex_hbm = refs[33]
    (s_sh, out_sh, el0, el1, er0, er1, sx0, sx1, f0, f1,
     srco0, srco1, dsto0, dsto1, dst0, dst1, z16, z64,
     sA0, sA1, sB0, sB1, sF0, sF1, sE0, sE1,
     sIa0, sIb0, sIa1, sIb1) = refs[34:]

    c = lax.axis_index("c")
    s = lax.axis_index("s")
    base_e = s * _TILE_E
    noff = c * _N
    zv = jnp.zeros((16,), jnp.float32)
    idx0 = jnp.zeros((16,), jnp.int32)
    idx1 = jnp.ones((16,), jnp.int32)
    pairs = _NCHUNK // 2

    def zfill16(k, _):
        z16[k] = zv
        return 0
    lax.fori_loop(0, 200, zfill16, 0)

    def zfill64(k, _):
        for q in range(4):
            z64[k, pl.ds(q * 16, 16)] = zv
        return 0
    lax.fori_loop(0, 100, zfill64, 0)

    def zero_out(q, _):
        pltpu.sync_copy(z64, out_sh.at[pl.ds(s * _SLAB + q * 100, 100)])
        return 0

    def zero_s(q, _):
        pltpu.sync_copy(z16, s_sh.at[pl.ds(s * _SLAB + q * 200, 200)])
        return 0

    lanes = lax.iota(jnp.int32, 16)

    def make_exrow(elb, erb, sxb, off, idxp, idxq):
        def exrow(i, _):
            for u in range(4):
                k = i * 4 + u
                row = (_lane_bcast(elb[k], idxp) +
                       _lane_bcast(erb[k], idxq))
                row = jnp.where(row > 0, row, 0.2 * row)
                row = jnp.exp(row)
                m = jnp.where(off + k < _E, 1.0, 0.0)
                sxb[k] = row * m
            return 0
        return exrow

    def make_msgrow(exb, srowb, fb):
        def msgrow(i, _):
            for u in range(4):
                k = i * 4 + u
                coef = exb[k] / (srowb[k] + 1e-9)
                c0 = _lane_bcast(coef, idx0)
                c1 = _lane_bcast(coef, idx1)
                for q in range(4):
                    sl = pl.ds(q * 16, 16)
                    fb[k, sl] = fb[k, sl] * (c0 if q < 2 else c1)
            return 0
        return msgrow

    for g, (dt, rels) in enumerate(_GROUPS):
        lax.fori_loop(0, _SLAB // 100, zero_out, 0)
        plsc.subcore_barrier()
        for r in rels:
            sti, lp, dti, lq = _REL_LANES[r]
            es_r, ed_r, el_r, er_r, fs_r = (es_t[r], ed_t[r], pk_t[sti],
                                            pk_t[dti], fs_t[r])
            idxp = jnp.where(lanes < 2, lanes + lp, lp)
            idxq = jnp.where(lanes < 2, lanes + lq, lq)
            lax.fori_loop(0, _SLAB // 200, zero_s, 0)
            plsc.subcore_barrier()

            def idx_issue(off, srco, dstb, sia, sib):
                pltpu.async_copy(es_r.at[pl.ds(off, _K)], srco, sia)
                pltpu.async_copy(ed_r.at[pl.ds(off, _K)], dstb, sib)

            def idx_wait(srco, dstb, sia, sib):
                pltpu.make_async_copy(es_r.at[pl.ds(base_e, _K)], srco,
                                      sia).wait()
                pltpu.make_async_copy(ed_r.at[pl.ds(base_e, _K)], dstb,
                                      sib).wait()

            def g1_issue(srco, dsto, dstb, elb, erb, sa, sb):
                def addoff(g_, _):
                    sl = pl.ds(g_ * 16, 16)
                    srco[sl] = srco[sl] + noff
                    dsto[sl] = dstb[sl] + noff
                    return 0
                lax.fori_loop(0, _K // 16, addoff, 0)
                pltpu.async_copy(el_r.at[srco], elb, sa)
                pltpu.async_copy(er_r.at[dsto], erb, sb)

            def g1_wait(srco, dsto, elb, erb, sa, sb):
                pltpu.make_async_copy(el_r.at[srco], elb, sa).wait()
                pltpu.make_async_copy(er_r.at[dsto], erb, sb).wait()

            idx_issue(base_e, srco0, dst0, sIa0, sIb0)
            idx_issue(base_e + _K, srco1, dst1, sIa1, sIb1)
            idx_wait(srco0, dst0, sIa0, sIb0)
            g1_issue(srco0, dsto0, dst0, el0, er0, sA0, sB0)
            idx_wait(srco1, dst1, sIa1, sIb1)
            g1_issue(srco1, dsto1, dst1, el1, er1, sA1, sB1)

            def p1_pair(i, _):
                off0 = base_e + (2 * i) * _K
                off1 = off0 + _K
                more = i + 1 < pairs
                g1_wait(srco0, dsto0, el0, er0, sA0, sB0)
                lax.fori_loop(0, _K // 4,
                              make_exrow(el0, er0, sx0, off0,
                                         idxp, idxq), 0)
                cpe0 = pltpu.async_copy(sx0, ex_hbm.at[c, pl.ds(off0, _K)],
                                        sE0)
                pltpu.sync_copy(sx0, s_sh.at[dst0], add=True)

                @pl.when(more)
                def _():
                    idx_issue(off0 + 2 * _K, srco0, dst0, sIa0, sIb0)
                g1_wait(srco1, dsto1, el1, er1, sA1, sB1)

                lax.fori_loop(0, _K // 4,
                              make_exrow(el1, er1, sx1, off1,
                                         idxp, idxq), 0)
                cpe1 = pltpu.async_copy(sx1, ex_hbm.at[c, pl.ds(off1, _K)],
                                        sE1)
                pltpu.sync_copy(sx1, s_sh.at[dst1], add=True)

                @pl.when(more)
                def _():
                    idx_issue(off1 + 2 * _K, srco1, dst1, sIa1, sIb1)
                    idx_wait(srco0, dst0, sIa0, sIb0)
                    g1_issue(srco0, dsto0, dst0, el0, er0, sA0, sB0)
                    idx_wait(srco1, dst1, sIa1, sIb1)
                    g1_issue(srco1, dsto1, dst1, el1, er1, sA1, sB1)
                cpe0.wait()
                cpe1.wait()
                return 0
            lax.fori_loop(0, pairs, p1_pair, 0)
            plsc.subcore_barrier()

            def g2_issue(off, srco, dstb, exb, srowb, fb, sa, sb, sf):
                def addoff(g_, _):
                    sl = pl.ds(g_ * 16, 16)
                    srco[sl] = srco[sl] + noff
                    return 0
                lax.fori_loop(0, _K // 16, addoff, 0)
                pltpu.async_copy(ex_hbm.at[c, pl.ds(off, _K)], exb, sa)
                pltpu.async_copy(s_sh.at[dstb], srowb, sb)
                pltpu.async_copy(fs_r.at[srco], fb, sf)

            def g2_wait(srco, dstb, exb, srowb, fb, sa, sb, sf):
                pltpu.make_async_copy(ex_hbm.at[c, pl.ds(base_e, _K)], exb,
                                      sa).wait()
                pltpu.make_async_copy(s_sh.at[dstb], srowb, sb).wait()
                pltpu.make_async_copy(fs_r.at[srco], fb, sf).wait()

            idx_issue(base_e, srco0, dst0, sIa0, sIb0)
            idx_issue(base_e + _K, srco1, dst1, sIa1, sIb1)
            idx_wait(srco0, dst0, sIa0, sIb0)
            g2_issue(base_e, srco0, dst0, el0, er0, f0, sA0, sB0, sF0)
            idx_wait(srco1, dst1, sIa1, sIb1)
            g2_issue(base_e + _K, srco1, dst1, el1, er1, f1, sA1, sB1, sF1)

            def p2_pair(i, _):
                off0 = base_e + (2 * i) * _K
                off1 = off0 + _K
                more = i + 1 < pairs
                g2_wait(srco0, dst0, el0, er0, f0, sA0, sB0, sF0)
                lax.fori_loop(0, _K // 4, make_msgrow(el0, er0, f0), 0)
                pltpu.sync_copy(f0, out_sh.at[dst0], add=True)

                @pl.when(more)
                def _():
                    idx_issue(off0 + 2 * _K, srco0, dst0, sIa0, sIb0)
                g2_wait(srco1, dst1, el1, er1, f1, sA1, sB1, sF1)

                lax.fori_loop(0, _K // 4, make_msgrow(el1, er1, f1), 0)
                pltpu.sync_copy(f1, out_sh.at[dst1], add=True)

                @pl.when(more)
                def _():
                    idx_issue(off1 + 2 * _K, srco1, dst1, sIa1, sIb1)
                    idx_wait(srco0, dst0, sIa0, sIb0)
                    g2_issue(off0 + 2 * _K, srco0, dst0, el0, er0, f0,
                             sA0, sB0, sF0)
                    idx_wait(srco1, dst1, sIa1, sIb1)
                    g2_issue(off1 + 2 * _K, srco1, dst1, el1, er1, f1,
                             sA1, sB1, sF1)
                return 0
            lax.fori_loop(0, pairs, p2_pair, 0)
            plsc.subcore_barrier()

        pltpu.sync_copy(out_sh.at[pl.ds(s * _SLAB, _SLAB)],
                        out_hbms[g].at[c, pl.ds(s * _SLAB, _SLAB)])
        plsc.subcore_barrier()


@functools.lru_cache(maxsize=None)
def _sc_layer_call():
    mesh = plsc.VectorSubcoreMesh(core_axis_name="c", subcore_axis_name="s")
    return pl.kernel(
        _sc_layer_body,
        out_type=[jax.ShapeDtypeStruct((2, _N, 64), jnp.float32)] * 3 +
                 [jax.ShapeDtypeStruct((2, _EPAD, 16), jnp.float32)],
        mesh=mesh,
        compiler_params=pltpu.CompilerParams(use_tc_tiling_on_sc=False),
        scratch_types=[
            pltpu.VMEM_SHARED((_N, 16), jnp.float32),
            pltpu.VMEM_SHARED((_N, 64), jnp.float32),
            pltpu.VMEM((_K, 16), jnp.float32),
            pltpu.VMEM((_K, 16), jnp.float32),
            pltpu.VMEM((_K, 16), jnp.float32),
            pltpu.VMEM((_K, 16), jnp.float32),
            pltpu.VMEM((_K, 16), jnp.float32),
            pltpu.VMEM((_K, 16), jnp.float32),
            pltpu.VMEM((_K, 64), jnp.float32),
            pltpu.VMEM((_K, 64), jnp.float32),
            pltpu.VMEM((_K,), jnp.int32),
            pltpu.VMEM((_K,), jnp.int32),
            pltpu.VMEM((_K,), jnp.int32),
            pltpu.VMEM((_K,), jnp.int32),
            pltpu.VMEM((_K,), jnp.int32),
            pltpu.VMEM((_K,), jnp.int32),
            pltpu.VMEM((200, 16), jnp.float32),
            pltpu.VMEM((100, 64), jnp.float32),
        ] + [pltpu.SemaphoreType.DMA] * 12,
    )


def _edge_phase(fsrc, elerpk, esrc, edst):
    outs = _sc_layer_call()(*esrc, *edst, *elerpk, *fsrc)
    return {dt: outs[g] for g, (dt, _) in enumerate(_GROUPS)}


def kernel(x_assmpt, x_rule, x_non_assmpt,
           e0, e1, e2, e3, e4, e5, e6, e7, e8,
           W_emb_a, b_emb_a, W_emb_r, b_emb_r, W_emb_n, b_emb_n,
           fc_W0, fc_b0, al0, ar0,
           fc_W1, fc_b1, al1, ar1,
           fc_W2, fc_b2, al2, ar2,
           ln_g_a, ln_b_a, cls_W_a, cls_b_a,
           ln_g_r, ln_b_r, cls_W_r, cls_b_r,
           ln_g_n, ln_b_n, cls_W_n, cls_b_n):
    xs = {'a': x_assmpt, 'r': x_rule, 'n': x_non_assmpt}
    wemb = {'a': (W_emb_a, b_emb_a), 'r': (W_emb_r, b_emb_r),
            'n': (W_emb_n, b_emb_n)}
    lng = {'a': (ln_g_a, ln_b_a), 'r': (ln_g_r, ln_b_r), 'n': (ln_g_n, ln_b_n)}
    cls = {'a': (cls_W_a, cls_b_a), 'r': (cls_W_r, cls_b_r),
           'n': (cls_W_n, cls_b_n)}
    fcw = (fc_W0, fc_W1, fc_W2)
    fcb = (fc_b0, fc_b1, fc_b2)
    als = (al0, al1, al2)
    ars = (ar0, ar1, ar2)
    edges = (e0, e1, e2, e3, e4, e5, e6, e7, e8)

    zpad = jnp.zeros((_EPAD - _E,), jnp.int32)
    esrc = [jnp.concatenate([e[0], zpad]) for e in edges]
    edst = [jnp.concatenate([e[1], zpad]) for e in edges]

    h = {t: _mm(xs[t], wemb[t][0].T) + wemb[t][1] for t in _TYPES}

    for l in range(3):
        fsrc, elerpk = [None] * 9, [None] * 3
        for t in _TYPES:
            pieces, fcols, lcols, rcols = [], [], [], []
            w = 0
            for i in _SRC_RELS[t]:
                wt = fcw[l][i].T
                fcols.append(w)
                pieces.append(wt)
                w += _HID
                lcols.append(w)
                pieces.append(wt @ _head_mat(als[l][i]))
                w += _HEADS
            for i in _DST_RELS[t]:
                rcols.append(w)
                pieces.append(fcw[l][i].T @ _head_mat(ars[l][i]))
                w += _HEADS
            outs = _proj(h[t], jnp.concatenate(pieces, axis=1),
                         tuple(fcols), tuple(lcols), tuple(rcols))
            for i, o in zip(_SRC_RELS[t], outs[:-1]):
                fsrc[i] = o.reshape(2 * _N, 64)
            elerpk[_TIDX[t]] = outs[-1].reshape(2 * _N, 16)

        acc = _edge_phase(fsrc, elerpk, esrc, edst)

        bias_sums = {t: jnp.zeros((_HID,), jnp.float32) for t in _TYPES}
        for i, (st, dt) in enumerate(_EL):
            bias_sums[dt] = bias_sums[dt] + fcb[l][i]
        h = {t: _ln_relu(acc[t], lng[t][0], lng[t][1], bias_sums[t])
             for t in _TYPES}

    outs = []
    for t in _TYPES:
        w = jnp.concatenate(
            [cls[t][0].T, jnp.zeros((_HID, 8 - cls[t][0].shape[0]),
                                    jnp.float32)], axis=1)
        o = _mm(h[t], w)[:, :cls[t][0].shape[0]] + cls[t][1]
        outs.append(o)
    return tuple(outs)
